# Initial kernel scaffold; baseline (speedup 1.0000x reference)
import jax
import jax.numpy as jnp
from jax.experimental import pallas as pl


def kernel(x, edge_index, W1, b1, g1, be1, rm1, rv1, W2, b2, g2, be2, rm2, rv2):
    raise NotImplementedError("write your pallas kernel here")


# speedup vs baseline: 13.4177x; 13.4177x over previous
import functools

import jax
import jax.numpy as jnp
from jax import lax
from jax.experimental import pallas as pl
from jax.experimental.pallas import tpu as pltpu
from jax.experimental.pallas import tpu_sc as plsc

N_NODES = 10000
IN_DIM = 128
HID = 64
EPS = 1e-5

NC = 2
NS = 16
NW = NC * NS

E = 320000
K = 128
NCHUNK = 80
EPT = K * NCHUNK
E_PAD = EPT * NW
N_PAD = 10240
ROWS_PT = N_PAD // NS
DDEG = 16

BM = 1024


def _sc_mesh():
    return plsc.VectorSubcoreMesh(core_axis_name="c", subcore_axis_name="s")


@functools.partial(
    pl.kernel,
    mesh=_sc_mesh(),
    compiler_params=pltpu.CompilerParams(use_tc_tiling_on_sc=False),
    out_type=jax.ShapeDtypeStruct((NC, N_PAD, DDEG), jnp.float32),
    scratch_types=[
        pltpu.VMEM((NCHUNK, K), jnp.int32),
        pltpu.VMEM((K, DDEG), jnp.float32),
        pltpu.VMEM_SHARED((N_PAD, DDEG), jnp.float32),
    ],
)
def _deg_kernel(dst_hbm, ones_hbm, zeros_hbm, out_hbm, dst_v, ones_v, acc):
    cid = lax.axis_index("c")
    sid = lax.axis_index("s")
    wid = sid * NC + cid
    base = sid * ROWS_PT
    pltpu.sync_copy(zeros_hbm.at[pl.ds(base, ROWS_PT)],
                    acc.at[pl.ds(base, ROWS_PT)])
    pltpu.sync_copy(dst_hbm.at[wid], dst_v)
    pltpu.sync_copy(ones_hbm, ones_v)
    plsc.subcore_barrier()

    @pl.loop(0, NCHUNK)
    def _(c):
        pltpu.sync_copy(ones_v, acc.at[dst_v.at[c]], add=True)

    plsc.subcore_barrier()
    pltpu.sync_copy(acc.at[pl.ds(base, ROWS_PT)],
                    out_hbm.at[cid, pl.ds(base, ROWS_PT)])


@functools.partial(
    pl.kernel,
    mesh=_sc_mesh(),
    compiler_params=pltpu.CompilerParams(use_tc_tiling_on_sc=False),
    out_type=jax.ShapeDtypeStruct((NC, N_PAD, HID), jnp.float32),
    scratch_types=[
        pltpu.VMEM((NCHUNK, K), jnp.int32),
        pltpu.VMEM((NCHUNK, K), jnp.int32),
        pltpu.VMEM((K, HID), jnp.float32),
        pltpu.VMEM_SHARED((N_PAD, HID), jnp.float32),
        pltpu.SemaphoreType.DMA,
    ],
)
def _msg_kernel(src_hbm, dst_hbm, tbl_hbm, zeros_hbm, out_hbm,
                src_v, dst_v, buf, acc, sem):
    cid = lax.axis_index("c")
    sid = lax.axis_index("s")
    wid = sid * NC + cid
    base = sid * ROWS_PT
    pltpu.sync_copy(zeros_hbm.at[pl.ds(base, ROWS_PT)],
                    acc.at[pl.ds(base, ROWS_PT)])
    pltpu.sync_copy(src_hbm.at[wid], src_v)
    pltpu.sync_copy(dst_hbm.at[wid], dst_v)
    plsc.subcore_barrier()

    @pl.loop(0, NCHUNK)
    def _(c):
        pltpu.async_copy(tbl_hbm.at[src_v.at[c]], buf, sem).wait()
        pltpu.sync_copy(buf, acc.at[dst_v.at[c]], add=True)

    plsc.subcore_barrier()
    pltpu.sync_copy(acc.at[pl.ds(base, ROWS_PT)],
                    out_hbm.at[cid, pl.ds(base, ROWS_PT)])


def _mm_body(x_ref, w_ref, o_ref):
    o_ref[...] = jnp.dot(x_ref[...], w_ref[...],
                         preferred_element_type=jnp.float32)


def _matmul(x, w, bm):
    n, kdim = x.shape
    d = w.shape[1]
    return pl.pallas_call(
        _mm_body,
        grid=(n // bm,),
        in_specs=[pl.BlockSpec((bm, kdim), lambda i: (i, 0)),
                  pl.BlockSpec((kdim, d), lambda i: (0, 0))],
        out_specs=pl.BlockSpec((bm, d), lambda i: (i, 0)),
        out_shape=jax.ShapeDtypeStruct((n, d), jnp.float32),
    )(x, w)


def _scale_body(wx_ref, deg_ref, o_wxs, o_dinv):
    deg = deg_ref[0, :, 0:1] + deg_ref[1, :, 0:1] + 1.0
    dinv = lax.rsqrt(deg)
    o_wxs[...] = wx_ref[...] * dinv
    o_dinv[...] = jnp.broadcast_to(dinv, o_dinv.shape)


def _scale1(wx, deg):
    return pl.pallas_call(
        _scale_body,
        grid=(N_PAD // BM,),
        in_specs=[pl.BlockSpec((BM, HID), lambda i: (i, 0)),
                  pl.BlockSpec((NC, BM, DDEG), lambda i: (0, i, 0))],
        out_specs=[pl.BlockSpec((BM, HID), lambda i: (i, 0)),
                   pl.BlockSpec((BM, HID), lambda i: (i, 0))],
        out_shape=[jax.ShapeDtypeStruct((N_PAD, HID), jnp.float32),
                   jax.ShapeDtypeStruct((N_PAD, HID), jnp.float32)],
    )(wx, deg)


def _l2_body(acc_ref, wxs_ref, dinv_ref, w2_ref, b_ref, g_ref, be_ref,
             rm_ref, rv_ref, o_ref):
    dinv = dinv_ref[...]
    g = dinv * (acc_ref[0] + acc_ref[1] + wxs_ref[...]) + b_ref[...]
    s = g_ref[...] * lax.rsqrt(rv_ref[...] + EPS)
    h = jnp.maximum((g - rm_ref[...]) * s + be_ref[...], 0.0)
    o_ref[...] = jnp.dot(h, w2_ref[...],
                         preferred_element_type=jnp.float32) * dinv


def _layer2_in(acc, wxs, dinv, w2, b, g, be, rm, rv):
    vec = pl.BlockSpec((1, HID), lambda i: (0, 0))
    blk = pl.BlockSpec((BM, HID), lambda i: (i, 0))
    return pl.pallas_call(
        _l2_body,
        grid=(N_PAD // BM,),
        in_specs=[pl.BlockSpec((NC, BM, HID), lambda i: (0, i, 0)),
                  blk, blk,
                  pl.BlockSpec((HID, HID), lambda i: (0, 0)),
                  vec, vec, vec, vec, vec],
        out_specs=blk,
        out_shape=jax.ShapeDtypeStruct((N_PAD, HID), jnp.float32),
    )(acc, wxs, dinv, w2, b, g, be, rm, rv)


def _fin_body(acc_ref, wxs_ref, dinv_ref, b_ref, g_ref, be_ref,
              rm_ref, rv_ref, o_ref):
    g = dinv_ref[...] * (acc_ref[0] + acc_ref[1] + wxs_ref[...]) + b_ref[...]
    s = g_ref[...] * lax.rsqrt(rv_ref[...] + EPS)
    o_ref[...] = jnp.maximum((g - rm_ref[...]) * s + be_ref[...], 0.0)


def _final(acc, wxs, dinv, b, g, be, rm, rv):
    vec = pl.BlockSpec((1, HID), lambda i: (0, 0))
    blk = pl.BlockSpec((BM, HID), lambda i: (i, 0))
    return pl.pallas_call(
        _fin_body,
        grid=(N_PAD // BM,),
        in_specs=[pl.BlockSpec((NC, BM, HID), lambda i: (0, i, 0)),
                  blk, blk, vec, vec, vec, vec, vec],
        out_specs=blk,
        out_shape=jax.ShapeDtypeStruct((N_PAD, HID), jnp.float32),
    )(acc, wxs, dinv, b, g, be, rm, rv)


def kernel(x, edge_index, W1, b1, g1, be1, rm1, rv1,
           W2, b2, g2, be2, rm2, rv2):
    src = edge_index[0].astype(jnp.int32)
    dst = edge_index[1].astype(jnp.int32)
    pad = jnp.full((E_PAD - E,), N_NODES, dtype=jnp.int32)
    src3 = jnp.concatenate([src, pad]).reshape(NW, NCHUNK, K)
    dst3 = jnp.concatenate([dst, pad]).reshape(NW, NCHUNK, K)

    x_pad = jnp.zeros((N_PAD, IN_DIM), jnp.float32).at[:N_NODES].set(x)
    zeros_deg = jnp.zeros((N_PAD, DDEG), jnp.float32)
    zeros_hid = jnp.zeros((N_PAD, HID), jnp.float32)
    ones_blk = jnp.ones((K, DDEG), jnp.float32)

    b1r, g1r, be1r, rm1r, rv1r = (v.reshape(1, HID) for v in (b1, g1, be1, rm1, rv1))
    b2r, g2r, be2r, rm2r, rv2r = (v.reshape(1, HID) for v in (b2, g2, be2, rm2, rv2))

    deg = _deg_kernel(dst3, ones_blk, zeros_deg)
    wx1 = _matmul(x_pad, W1, BM)
    wxs1, dinv = _scale1(wx1, deg)
    acc1 = _msg_kernel(src3, dst3, wxs1, zeros_hid)
    wxs2 = _layer2_in(acc1, wxs1, dinv, W2, b1r, g1r, be1r, rm1r, rv1r)
    acc2 = _msg_kernel(src3, dst3, wxs2, zeros_hid)
    out = _final(acc2, wxs2, dinv, b2r, g2r, be2r, rm2r, rv2r)
    return out[:N_NODES]
